# Initial kernel scaffold; baseline (speedup 1.0000x reference)
import jax
import jax.numpy as jnp
from jax.experimental import pallas as pl


def kernel(x, edge_index, W1, b1, W2, b2, g1_gamma, g1_beta, g2_gamma, g2_beta, n1_gamma, n1_beta):
    raise NotImplementedError("write your pallas kernel here")


# speedup vs baseline: 10.0805x; 10.0805x over previous
import functools
import jax
import jax.numpy as jnp
from jax import lax
from jax.experimental import pallas as pl
from jax.experimental.pallas import tpu as pltpu
from jax.experimental.pallas import tpu_sc as plsc

N_NODES = 10000
N_EDGES = 320000
IN_DIM = 128
G1 = 10
G2 = 10
BATCH = 4
D = BATCH * G1
NROW = 10112
NC = 2
NS = 16
NW = NC * NS
CH = 128
NCH = 80
EPW = CH * NCH
E_PAD = EPW * NW
ROWS_PER_TILE = NROW // NS
EPS = 1e-5

def _deg_body(src_hbm, dst_hbm, out_hbm, sv, dv, hs, hd):
    w = lax.axis_index("s") * NC + lax.axis_index("c")
    base = w * EPW
    pltpu.sync_copy(src_hbm.at[pl.ds(base, EPW)], sv)
    pltpu.sync_copy(dst_hbm.at[pl.ds(base, EPW)], dv)

    zeros16 = jnp.zeros((16,), jnp.float32)

    def zero_body(i, _):
        hs[pl.ds(i * 16, 16)] = zeros16
        hd[pl.ds(i * 16, 16)] = zeros16
        return 0

    lax.fori_loop(0, NROW // 16, zero_body, 0, unroll=4)

    ones16 = jnp.ones((16,), jnp.float32)

    def hist_body(i, _):
        s16 = sv[pl.ds(i * 16, 16)]
        plsc.addupdate_scatter(hs, [s16], ones16)
        d16 = dv[pl.ds(i * 16, 16)]
        plsc.addupdate_scatter(hd, [d16], ones16)
        return 0

    lax.fori_loop(0, EPW // 16, hist_body, 0, unroll=4)

    pltpu.sync_copy(hs, out_hbm.at[w, 0])
    pltpu.sync_copy(hd, out_hbm.at[w, 1])


def _edge_body(table_hbm, src_hbm, dst_hbm, zeros_hbm, out_hbm,
               acc, sv, dv, rb0, rb1, sem0, sem1):
    c = lax.axis_index("c")
    s = lax.axis_index("s")
    w = s * NC + c

    pltpu.sync_copy(zeros_hbm.at[pl.ds(s * ROWS_PER_TILE, ROWS_PER_TILE)],
                    acc.at[pl.ds(s * ROWS_PER_TILE, ROWS_PER_TILE)])
    plsc.subcore_barrier()

    pltpu.sync_copy(src_hbm.at[w], sv)
    pltpu.sync_copy(dst_hbm.at[w], dv)

    pltpu.async_copy(table_hbm.at[sv.at[0]], rb0, sem0)

    def body(i, _):
        k0 = 2 * i
        pltpu.make_async_copy(table_hbm.at[sv.at[k0]], rb0, sem0).wait()
        pltpu.async_copy(table_hbm.at[sv.at[k0 + 1]], rb1, sem1)
        pltpu.sync_copy(rb0, acc.at[dv.at[k0]], add=True)
        pltpu.make_async_copy(table_hbm.at[sv.at[k0 + 1]], rb1, sem1).wait()

        @pl.when(k0 + 2 < NCH)
        def _():
            pltpu.async_copy(table_hbm.at[sv.at[k0 + 2]], rb0, sem0)

        pltpu.sync_copy(rb1, acc.at[dv.at[k0 + 1]], add=True)
        return 0

    lax.fori_loop(0, NCH // 2, body, 0)

    plsc.subcore_barrier()
    pltpu.sync_copy(acc.at[pl.ds(s * ROWS_PER_TILE, ROWS_PER_TILE)],
                    out_hbm.at[c, pl.ds(s * ROWS_PER_TILE, ROWS_PER_TILE)])


@functools.lru_cache(maxsize=None)
def _sc_kernels():
    mesh = plsc.VectorSubcoreMesh(core_axis_name="c", subcore_axis_name="s",
                                  num_cores=NC, num_subcores=NS)
    sc_params = pltpu.CompilerParams(needs_layout_passes=False,
                                     use_tc_tiling_on_sc=False)
    deg_kernel = pl.kernel(
        _deg_body,
        out_type=jax.ShapeDtypeStruct((NW, 2, NROW), jnp.float32),
        mesh=mesh,
        compiler_params=sc_params,
        scratch_types=[
            pltpu.VMEM((EPW,), jnp.int32),
            pltpu.VMEM((EPW,), jnp.int32),
            pltpu.VMEM((NROW,), jnp.float32),
            pltpu.VMEM((NROW,), jnp.float32),
        ],
    )
    edge_kernel = pl.kernel(
        _edge_body,
        out_type=jax.ShapeDtypeStruct((NC, NROW, D), jnp.float32),
        mesh=mesh,
        compiler_params=sc_params,
        scratch_types=[
            pltpu.VMEM_SHARED((NROW, D), jnp.float32),
            pltpu.VMEM((NCH, CH), jnp.int32),
            pltpu.VMEM((NCH, CH), jnp.int32),
            pltpu.VMEM((CH, D), jnp.float32),
            pltpu.VMEM((CH, D), jnp.float32),
            pltpu.SemaphoreType.DMA,
            pltpu.SemaphoreType.DMA,
        ],
    )
    return deg_kernel, edge_kernel


def _proj_body(xr_ref, w1_ref, degp_ref, out_ref):
    deg = jnp.sum(degp_ref[...], axis=0)
    nsrc = lax.rsqrt(jnp.maximum(deg[0], 1.0))
    w1 = w1_ref[...]
    outs = []
    for b in range(BATCH):
        xb = xr_ref[b] * nsrc[None, :]
        r = lax.dot_general(xb, w1, (((0,), (0,)), ((), ())),
                            preferred_element_type=jnp.float32)
        outs.append(r)
    out_ref[...] = jnp.concatenate(outs, axis=1)


_NB = 2048


def _project(xr, W1, degp):
    return pl.pallas_call(
        _proj_body,
        grid=(5,),
        in_specs=[
            pl.BlockSpec((BATCH, IN_DIM, _NB), lambda i: (0, 0, i)),
            pl.BlockSpec((IN_DIM, G1), lambda i: (0, 0)),
            pl.BlockSpec((NW, 2, _NB), lambda i: (0, 0, i)),
        ],
        out_specs=pl.BlockSpec((_NB, D), lambda i: (i, 0)),
        out_shape=jax.ShapeDtypeStruct((NROW, D), jnp.float32),
    )(xr, W1, degp)


def _colsum(h):
    n = h.shape[0]
    p = 1 << (n - 1).bit_length()
    if p != n:
        h = jnp.concatenate(
            [h, jnp.zeros((p - n,) + h.shape[1:], h.dtype)], axis=0)
    while p > 8:
        p //= 2
        h = h[:p] + h[p:]
    return jnp.sum(h, axis=0)


def _mid_body(parts_ref, degp_ref, w2b_ref, b1t_ref, g1g_ref, g1b_ref, out_ref):
    a = parts_ref[0, :N_NODES] + parts_ref[1, :N_NODES]
    deg = jnp.sum(degp_ref[...], axis=0)
    nsrc = lax.rsqrt(jnp.maximum(deg[0, :N_NODES], 1.0))
    ndst = lax.rsqrt(jnp.maximum(deg[1, :N_NODES], 1.0))
    h = a * ndst[:, None] + b1t_ref[...]
    mean = _colsum(h) * (1.0 / N_NODES)
    cen = h - mean
    var = _colsum(cen * cen) * (1.0 / N_NODES)
    hn = jax.nn.relu(g1g_ref[...] * cen / jnp.sqrt(var + EPS) + g1b_ref[...])
    hn = hn * nsrc[:, None]
    h2 = lax.dot_general(hn, w2b_ref[...], (((1,), (0,)), ((), ())),
                         preferred_element_type=jnp.float32)
    out_ref[...] = jnp.concatenate(
        [h2, jnp.zeros((NROW - N_NODES, D), jnp.float32)], axis=0)


def _mid(parts, degp, W2blk, b1t, g1gt, g1bt):
    return pl.pallas_call(
        _mid_body,
        out_shape=jax.ShapeDtypeStruct((NROW, D), jnp.float32),
    )(parts, degp, W2blk, b1t, g1gt, g1bt)


def _final_body(parts_ref, degp_ref, b2t_ref, g2g_ref, g2b_ref,
                n1g_ref, n1b_ref, u_ref, v_ref, out_ref):
    a = parts_ref[0, :N_NODES] + parts_ref[1, :N_NODES]
    deg = jnp.sum(degp_ref[...], axis=0)
    ndst = lax.rsqrt(jnp.maximum(deg[1, :N_NODES], 1.0))
    h = a * ndst[:, None] + b2t_ref[...]
    mean = _colsum(h) * (1.0 / N_NODES)
    cen = h - mean
    var = _colsum(cen * cen) * (1.0 / N_NODES)
    hn = jax.nn.relu(g2g_ref[...] * cen / jnp.sqrt(var + EPS) + g2b_ref[...])
    m = _colsum(hn) * (1.0 / N_NODES)
    mb = lax.dot_general(u_ref[...] * m[None, :], v_ref[...],
                         (((1,), (0,)), ((), ())),
                         precision=lax.Precision.HIGHEST,
                         preferred_element_type=jnp.float32)
    mu = jnp.mean(mb, axis=0)
    cb = mb - mu
    vb = jnp.mean(cb * cb, axis=0)
    out_ref[...] = n1g_ref[...] * cb / jnp.sqrt(vb + EPS) + n1b_ref[...]


def _final(parts, degp, b2t, g2gt, g2bt, n1_gamma, n1_beta):
    ks = jnp.arange(D)
    u = (ks[None, :] // G2 == jnp.arange(BATCH)[:, None]).astype(jnp.float32)
    v = (ks[:, None] % G2 == jnp.arange(G2)[None, :]).astype(jnp.float32)
    return pl.pallas_call(
        _final_body,
        out_shape=jax.ShapeDtypeStruct((BATCH, G2), jnp.float32),
    )(parts, degp, b2t, g2gt, g2bt, n1_gamma, n1_beta, u, v)


def kernel(x, edge_index, W1, b1, W2, b2, g1_gamma, g1_beta,
           g2_gamma, g2_beta, n1_gamma, n1_beta):
    src = edge_index[0]
    dst = edge_index[1]
    pad = jnp.full((E_PAD - N_EDGES,), N_NODES, jnp.int32)
    src_p = jnp.concatenate([src, pad])
    dst_p = jnp.concatenate([dst, pad])
    src2d = src_p.reshape(NW, NCH, CH)
    dst2d = dst_p.reshape(NW, NCH, CH)

    xr = x.reshape(BATCH, IN_DIM, N_NODES)

    deg_kernel, edge_kernel = _sc_kernels()
    degp = deg_kernel(src_p, dst_p)
    table1 = _project(xr, W1, degp)

    zrs = jnp.zeros((NROW, D), jnp.float32)
    parts1 = edge_kernel(table1, src2d, dst2d, zrs)

    eye = jnp.eye(BATCH, dtype=jnp.float32)
    W2blk = jnp.kron(eye, W2)
    b1t = jnp.tile(b1, BATCH)
    g1gt = jnp.tile(g1_gamma, BATCH)
    g1bt = jnp.tile(g1_beta, BATCH)
    table2 = _mid(parts1, degp, W2blk, b1t, g1gt, g1bt)

    parts2 = edge_kernel(table2, src2d, dst2d, zrs)

    b2t = jnp.tile(b2, BATCH)
    g2gt = jnp.tile(g2_gamma, BATCH)
    g2bt = jnp.tile(g2_beta, BATCH)
    return _final(parts2, degp, b2t, g2gt, g2bt, n1_gamma, n1_beta)


# speedup vs baseline: 10.5998x; 1.0515x over previous
import functools
import jax
import jax.numpy as jnp
from jax import lax
from jax.experimental import pallas as pl
from jax.experimental.pallas import tpu as pltpu
from jax.experimental.pallas import tpu_sc as plsc

N_NODES = 10000
N_EDGES = 320000
IN_DIM = 128
G1 = 10
G2 = 10
BATCH = 4
D = BATCH * G1
NROW = 10112
NC = 2
NS = 16
NW = NC * NS
CH = 128
NCH = 80
EPW = CH * NCH
E_PAD = EPW * NW
ROWS_PER_TILE = NROW // NS
EPS = 1e-5

def _deg_body(src_hbm, dst_hbm, out_hbm, sv, dv, hs, hd):
    w = lax.axis_index("s") * NC + lax.axis_index("c")
    base = w * EPW
    pltpu.sync_copy(src_hbm.at[pl.ds(base, EPW)], sv)
    pltpu.sync_copy(dst_hbm.at[pl.ds(base, EPW)], dv)

    zeros16 = jnp.zeros((16,), jnp.float32)

    def zero_body(i, _):
        hs[pl.ds(i * 16, 16)] = zeros16
        hd[pl.ds(i * 16, 16)] = zeros16
        return 0

    lax.fori_loop(0, NROW // 16, zero_body, 0, unroll=4)

    ones16 = jnp.ones((16,), jnp.float32)

    def hist_body(i, _):
        s16 = sv[pl.ds(i * 16, 16)]
        plsc.addupdate_scatter(hs, [s16], ones16)
        d16 = dv[pl.ds(i * 16, 16)]
        plsc.addupdate_scatter(hd, [d16], ones16)
        return 0

    lax.fori_loop(0, EPW // 16, hist_body, 0, unroll=4)

    pltpu.sync_copy(hs, out_hbm.at[w, 0])
    pltpu.sync_copy(hd, out_hbm.at[w, 1])


NSLOT = 4


def _edge_body(table_hbm, src_hbm, dst_hbm, zeros_hbm, out_hbm,
               acc, sv, dv, rb0, rb1, rb2, rb3,
               gs0, gs1, gs2, gs3, ss0, ss1, ss2, ss3):
    c = lax.axis_index("c")
    s = lax.axis_index("s")
    w = s * NC + c

    pltpu.sync_copy(zeros_hbm.at[pl.ds(s * ROWS_PER_TILE, ROWS_PER_TILE)],
                    acc.at[pl.ds(s * ROWS_PER_TILE, ROWS_PER_TILE)])
    plsc.subcore_barrier()

    pltpu.sync_copy(src_hbm.at[w], sv)
    pltpu.sync_copy(dst_hbm.at[w], dv)

    rbs = (rb0, rb1, rb2, rb3)
    gsems = (gs0, gs1, gs2, gs3)
    ssems = (ss0, ss1, ss2, ss3)
    for b in range(NSLOT):
        pltpu.async_copy(table_hbm.at[sv.at[b]], rbs[b], gsems[b])

    def body(i, _):
        k0 = NSLOT * i
        for b in range(NSLOT):
            k = k0 + b
            pltpu.make_async_copy(table_hbm.at[sv.at[k]], rbs[b],
                                  gsems[b]).wait()
            pltpu.async_copy(rbs[b], acc.at[dv.at[k]], ssems[b], add=True)
        for b in range(NSLOT):
            k = k0 + b
            pltpu.make_async_copy(rbs[b], acc.at[dv.at[k]], ssems[b]).wait()

            @pl.when(k + NSLOT < NCH)
            def _():
                pltpu.async_copy(table_hbm.at[sv.at[k + NSLOT]], rbs[b],
                                 gsems[b])
        return 0

    lax.fori_loop(0, NCH // NSLOT, body, 0)

    plsc.subcore_barrier()
    pltpu.sync_copy(acc.at[pl.ds(s * ROWS_PER_TILE, ROWS_PER_TILE)],
                    out_hbm.at[c, pl.ds(s * ROWS_PER_TILE, ROWS_PER_TILE)])


@functools.lru_cache(maxsize=None)
def _sc_kernels():
    mesh = plsc.VectorSubcoreMesh(core_axis_name="c", subcore_axis_name="s",
                                  num_cores=NC, num_subcores=NS)
    sc_params = pltpu.CompilerParams(needs_layout_passes=False,
                                     use_tc_tiling_on_sc=False)
    deg_kernel = pl.kernel(
        _deg_body,
        out_type=jax.ShapeDtypeStruct((NW, 2, NROW), jnp.float32),
        mesh=mesh,
        compiler_params=sc_params,
        scratch_types=[
            pltpu.VMEM((EPW,), jnp.int32),
            pltpu.VMEM((EPW,), jnp.int32),
            pltpu.VMEM((NROW,), jnp.float32),
            pltpu.VMEM((NROW,), jnp.float32),
        ],
    )
    edge_kernel = pl.kernel(
        _edge_body,
        out_type=jax.ShapeDtypeStruct((NC, NROW, D), jnp.float32),
        mesh=mesh,
        compiler_params=sc_params,
        scratch_types=[
            pltpu.VMEM_SHARED((NROW, D), jnp.float32),
            pltpu.VMEM((NCH, CH), jnp.int32),
            pltpu.VMEM((NCH, CH), jnp.int32),
        ] + [pltpu.VMEM((CH, D), jnp.float32)] * NSLOT
          + [pltpu.SemaphoreType.DMA] * (2 * NSLOT),
    )
    return deg_kernel, edge_kernel


def _proj_body(xr_ref, w1_ref, degp_ref, out_ref):
    deg = jnp.sum(degp_ref[...], axis=0)
    nsrc = lax.rsqrt(jnp.maximum(deg[0], 1.0))
    w1 = w1_ref[...]
    outs = []
    for b in range(BATCH):
        xb = xr_ref[b] * nsrc[None, :]
        r = lax.dot_general(xb, w1, (((0,), (0,)), ((), ())),
                            preferred_element_type=jnp.float32)
        outs.append(r)
    out_ref[...] = jnp.concatenate(outs, axis=1)


_NB = 2048


def _project(xr, W1, degp):
    return pl.pallas_call(
        _proj_body,
        grid=(5,),
        in_specs=[
            pl.BlockSpec((BATCH, IN_DIM, _NB), lambda i: (0, 0, i)),
            pl.BlockSpec((IN_DIM, G1), lambda i: (0, 0)),
            pl.BlockSpec((NW, 2, _NB), lambda i: (0, 0, i)),
        ],
        out_specs=pl.BlockSpec((_NB, D), lambda i: (i, 0)),
        out_shape=jax.ShapeDtypeStruct((NROW, D), jnp.float32),
    )(xr, W1, degp)


def _colsum(h):
    n = h.shape[0]
    p = 1 << (n - 1).bit_length()
    if p != n:
        h = jnp.concatenate(
            [h, jnp.zeros((p - n,) + h.shape[1:], h.dtype)], axis=0)
    while p > 8:
        p //= 2
        h = h[:p] + h[p:]
    return jnp.sum(h, axis=0)


def _mid_body(parts_ref, degp_ref, w2b_ref, b1t_ref, g1g_ref, g1b_ref, out_ref):
    a = parts_ref[0, :N_NODES] + parts_ref[1, :N_NODES]
    deg = jnp.sum(degp_ref[...], axis=0)
    nsrc = lax.rsqrt(jnp.maximum(deg[0, :N_NODES], 1.0))
    ndst = lax.rsqrt(jnp.maximum(deg[1, :N_NODES], 1.0))
    h = a * ndst[:, None] + b1t_ref[...]
    mean = _colsum(h) * (1.0 / N_NODES)
    cen = h - mean
    var = _colsum(cen * cen) * (1.0 / N_NODES)
    hn = jax.nn.relu(g1g_ref[...] * cen / jnp.sqrt(var + EPS) + g1b_ref[...])
    hn = hn * nsrc[:, None]
    h2 = lax.dot_general(hn, w2b_ref[...], (((1,), (0,)), ((), ())),
                         preferred_element_type=jnp.float32)
    out_ref[...] = jnp.concatenate(
        [h2, jnp.zeros((NROW - N_NODES, D), jnp.float32)], axis=0)


def _mid(parts, degp, W2blk, b1t, g1gt, g1bt):
    return pl.pallas_call(
        _mid_body,
        out_shape=jax.ShapeDtypeStruct((NROW, D), jnp.float32),
    )(parts, degp, W2blk, b1t, g1gt, g1bt)


def _final_body(parts_ref, degp_ref, b2t_ref, g2g_ref, g2b_ref,
                n1g_ref, n1b_ref, u_ref, v_ref, out_ref):
    a = parts_ref[0, :N_NODES] + parts_ref[1, :N_NODES]
    deg = jnp.sum(degp_ref[...], axis=0)
    ndst = lax.rsqrt(jnp.maximum(deg[1, :N_NODES], 1.0))
    h = a * ndst[:, None] + b2t_ref[...]
    mean = _colsum(h) * (1.0 / N_NODES)
    cen = h - mean
    var = _colsum(cen * cen) * (1.0 / N_NODES)
    hn = jax.nn.relu(g2g_ref[...] * cen / jnp.sqrt(var + EPS) + g2b_ref[...])
    m = _colsum(hn) * (1.0 / N_NODES)
    mb = lax.dot_general(u_ref[...] * m[None, :], v_ref[...],
                         (((1,), (0,)), ((), ())),
                         precision=lax.Precision.HIGHEST,
                         preferred_element_type=jnp.float32)
    mu = jnp.mean(mb, axis=0)
    cb = mb - mu
    vb = jnp.mean(cb * cb, axis=0)
    out_ref[...] = n1g_ref[...] * cb / jnp.sqrt(vb + EPS) + n1b_ref[...]


def _final(parts, degp, b2t, g2gt, g2bt, n1_gamma, n1_beta):
    ks = jnp.arange(D)
    u = (ks[None, :] // G2 == jnp.arange(BATCH)[:, None]).astype(jnp.float32)
    v = (ks[:, None] % G2 == jnp.arange(G2)[None, :]).astype(jnp.float32)
    return pl.pallas_call(
        _final_body,
        out_shape=jax.ShapeDtypeStruct((BATCH, G2), jnp.float32),
    )(parts, degp, b2t, g2gt, g2bt, n1_gamma, n1_beta, u, v)


def kernel(x, edge_index, W1, b1, W2, b2, g1_gamma, g1_beta,
           g2_gamma, g2_beta, n1_gamma, n1_beta):
    src = edge_index[0]
    dst = edge_index[1]
    pad = jnp.full((E_PAD - N_EDGES,), N_NODES, jnp.int32)
    src_p = jnp.concatenate([src, pad])
    dst_p = jnp.concatenate([dst, pad])
    src2d = src_p.reshape(NW, NCH, CH)
    dst2d = dst_p.reshape(NW, NCH, CH)

    xr = x.reshape(BATCH, IN_DIM, N_NODES)

    deg_kernel, edge_kernel = _sc_kernels()
    degp = deg_kernel(src_p, dst_p)
    table1 = _project(xr, W1, degp)

    zrs = jnp.zeros((NROW, D), jnp.float32)
    parts1 = edge_kernel(table1, src2d, dst2d, zrs)

    eye = jnp.eye(BATCH, dtype=jnp.float32)
    W2blk = jnp.kron(eye, W2)
    b1t = jnp.tile(b1, BATCH)
    g1gt = jnp.tile(g1_gamma, BATCH)
    g1bt = jnp.tile(g1_beta, BATCH)
    table2 = _mid(parts1, degp, W2blk, b1t, g1gt, g1bt)

    parts2 = edge_kernel(table2, src2d, dst2d, zrs)

    b2t = jnp.tile(b2, BATCH)
    g2gt = jnp.tile(g2_gamma, BATCH)
    g2bt = jnp.tile(g2_beta, BATCH)
    return _final(parts2, degp, b2t, g2gt, g2bt, n1_gamma, n1_beta)


# speedup vs baseline: 14.7125x; 1.3880x over previous
import functools
import jax
import jax.numpy as jnp
from jax import lax
from jax.experimental import pallas as pl
from jax.experimental.pallas import tpu as pltpu
from jax.experimental.pallas import tpu_sc as plsc

N_NODES = 10000
N_EDGES = 320000
IN_DIM = 128
G1 = 10
G2 = 10
BATCH = 4
D = BATCH * G1
NROW = 10112
NC = 2
NS = 16
NW = NC * NS
CH = 128
NCH = 80
EPW = CH * NCH
E_PAD = EPW * NW
ROWS_PER_TILE = NROW // NS
EPS = 1e-5

def _deg_body(src_hbm, dst_hbm, out_hbm, sv, dv, hs, hd):
    w = lax.axis_index("s") * NC + lax.axis_index("c")
    base = w * EPW
    pltpu.sync_copy(src_hbm.at[pl.ds(base, EPW)], sv)
    pltpu.sync_copy(dst_hbm.at[pl.ds(base, EPW)], dv)

    zeros16 = jnp.zeros((16,), jnp.float32)

    def zero_body(i, _):
        hs[pl.ds(i * 16, 16)] = zeros16
        hd[pl.ds(i * 16, 16)] = zeros16
        return 0

    lax.fori_loop(0, NROW // 16, zero_body, 0, unroll=4)

    ones16 = jnp.ones((16,), jnp.float32)

    def hist_body(i, _):
        s16 = sv[pl.ds(i * 16, 16)]
        plsc.addupdate_scatter(hs, [s16], ones16)
        d16 = dv[pl.ds(i * 16, 16)]
        plsc.addupdate_scatter(hd, [d16], ones16)
        return 0

    lax.fori_loop(0, EPW // 16, hist_body, 0, unroll=4)

    pltpu.sync_copy(hs, out_hbm.at[w, 0])
    pltpu.sync_copy(hd, out_hbm.at[w, 1])


NSLOT = 4


def _edge_body(table_hbm, src_hbm, dst_hbm, zeros_hbm, out_hbm,
               acc, tbl, sv, dv, rb0, rb1, rb2, rb3,
               gs0, gs1, gs2, gs3, ss0, ss1, ss2, ss3):
    c = lax.axis_index("c")
    s = lax.axis_index("s")
    w = s * NC + c

    pltpu.sync_copy(zeros_hbm.at[pl.ds(s * ROWS_PER_TILE, ROWS_PER_TILE)],
                    acc.at[pl.ds(s * ROWS_PER_TILE, ROWS_PER_TILE)])
    pltpu.sync_copy(table_hbm.at[pl.ds(s * ROWS_PER_TILE, ROWS_PER_TILE)],
                    tbl.at[pl.ds(s * ROWS_PER_TILE, ROWS_PER_TILE)])
    plsc.subcore_barrier()

    pltpu.sync_copy(src_hbm.at[w], sv)
    pltpu.sync_copy(dst_hbm.at[w], dv)

    rbs = (rb0, rb1, rb2, rb3)
    gsems = (gs0, gs1, gs2, gs3)
    ssems = (ss0, ss1, ss2, ss3)
    for b in range(NSLOT):
        pltpu.async_copy(tbl.at[sv.at[b]], rbs[b], gsems[b])

    def body(i, _):
        k0 = NSLOT * i
        for b in range(NSLOT):
            k = k0 + b
            pltpu.make_async_copy(tbl.at[sv.at[k]], rbs[b],
                                  gsems[b]).wait()
            pltpu.async_copy(rbs[b], acc.at[dv.at[k]], ssems[b], add=True)
        for b in range(NSLOT):
            k = k0 + b
            pltpu.make_async_copy(rbs[b], acc.at[dv.at[k]], ssems[b]).wait()

            @pl.when(k + NSLOT < NCH)
            def _():
                pltpu.async_copy(tbl.at[sv.at[k + NSLOT]], rbs[b],
                                 gsems[b])
        return 0

    lax.fori_loop(0, NCH // NSLOT, body, 0)

    plsc.subcore_barrier()
    pltpu.sync_copy(acc.at[pl.ds(s * ROWS_PER_TILE, ROWS_PER_TILE)],
                    out_hbm.at[c, pl.ds(s * ROWS_PER_TILE, ROWS_PER_TILE)])


@functools.lru_cache(maxsize=None)
def _sc_kernels():
    mesh = plsc.VectorSubcoreMesh(core_axis_name="c", subcore_axis_name="s",
                                  num_cores=NC, num_subcores=NS)
    sc_params = pltpu.CompilerParams(needs_layout_passes=False,
                                     use_tc_tiling_on_sc=False)
    deg_kernel = pl.kernel(
        _deg_body,
        out_type=jax.ShapeDtypeStruct((NW, 2, NROW), jnp.float32),
        mesh=mesh,
        compiler_params=sc_params,
        scratch_types=[
            pltpu.VMEM((EPW,), jnp.int32),
            pltpu.VMEM((EPW,), jnp.int32),
            pltpu.VMEM((NROW,), jnp.float32),
            pltpu.VMEM((NROW,), jnp.float32),
        ],
    )
    edge_kernel = pl.kernel(
        _edge_body,
        out_type=jax.ShapeDtypeStruct((NC, NROW, D), jnp.float32),
        mesh=mesh,
        compiler_params=sc_params,
        scratch_types=[
            pltpu.VMEM_SHARED((NROW, D), jnp.float32),
            pltpu.VMEM_SHARED((NROW, D), jnp.float32),
            pltpu.VMEM((NCH, CH), jnp.int32),
            pltpu.VMEM((NCH, CH), jnp.int32),
        ] + [pltpu.VMEM((CH, D), jnp.float32)] * NSLOT
          + [pltpu.SemaphoreType.DMA] * (2 * NSLOT),
    )
    return deg_kernel, edge_kernel


def _proj_body(xr_ref, w1_ref, degp_ref, out_ref):
    deg = jnp.sum(degp_ref[...], axis=0)
    nsrc = lax.rsqrt(jnp.maximum(deg[0], 1.0))
    w1 = w1_ref[...]
    outs = []
    for b in range(BATCH):
        xb = xr_ref[b] * nsrc[None, :]
        r = lax.dot_general(xb, w1, (((0,), (0,)), ((), ())),
                            preferred_element_type=jnp.float32)
        outs.append(r)
    out_ref[...] = jnp.concatenate(outs, axis=1)


_NB = 2048


def _project(xr, W1, degp):
    return pl.pallas_call(
        _proj_body,
        grid=(5,),
        in_specs=[
            pl.BlockSpec((BATCH, IN_DIM, _NB), lambda i: (0, 0, i)),
            pl.BlockSpec((IN_DIM, G1), lambda i: (0, 0)),
            pl.BlockSpec((NW, 2, _NB), lambda i: (0, 0, i)),
        ],
        out_specs=pl.BlockSpec((_NB, D), lambda i: (i, 0)),
        out_shape=jax.ShapeDtypeStruct((NROW, D), jnp.float32),
    )(xr, W1, degp)


def _colsum(h):
    n = h.shape[0]
    p = 1 << (n - 1).bit_length()
    if p != n:
        h = jnp.concatenate(
            [h, jnp.zeros((p - n,) + h.shape[1:], h.dtype)], axis=0)
    while p > 8:
        p //= 2
        h = h[:p] + h[p:]
    return jnp.sum(h, axis=0)


def _mid_body(parts_ref, degp_ref, w2b_ref, b1t_ref, g1g_ref, g1b_ref, out_ref):
    a = parts_ref[0, :N_NODES] + parts_ref[1, :N_NODES]
    deg = jnp.sum(degp_ref[...], axis=0)
    nsrc = lax.rsqrt(jnp.maximum(deg[0, :N_NODES], 1.0))
    ndst = lax.rsqrt(jnp.maximum(deg[1, :N_NODES], 1.0))
    h = a * ndst[:, None] + b1t_ref[...]
    mean = _colsum(h) * (1.0 / N_NODES)
    cen = h - mean
    var = _colsum(cen * cen) * (1.0 / N_NODES)
    hn = jax.nn.relu(g1g_ref[...] * cen / jnp.sqrt(var + EPS) + g1b_ref[...])
    hn = hn * nsrc[:, None]
    h2 = lax.dot_general(hn, w2b_ref[...], (((1,), (0,)), ((), ())),
                         preferred_element_type=jnp.float32)
    out_ref[...] = jnp.concatenate(
        [h2, jnp.zeros((NROW - N_NODES, D), jnp.float32)], axis=0)


def _mid(parts, degp, W2blk, b1t, g1gt, g1bt):
    return pl.pallas_call(
        _mid_body,
        out_shape=jax.ShapeDtypeStruct((NROW, D), jnp.float32),
    )(parts, degp, W2blk, b1t, g1gt, g1bt)


def _final_body(parts_ref, degp_ref, b2t_ref, g2g_ref, g2b_ref,
                n1g_ref, n1b_ref, u_ref, v_ref, out_ref):
    a = parts_ref[0, :N_NODES] + parts_ref[1, :N_NODES]
    deg = jnp.sum(degp_ref[...], axis=0)
    ndst = lax.rsqrt(jnp.maximum(deg[1, :N_NODES], 1.0))
    h = a * ndst[:, None] + b2t_ref[...]
    mean = _colsum(h) * (1.0 / N_NODES)
    cen = h - mean
    var = _colsum(cen * cen) * (1.0 / N_NODES)
    hn = jax.nn.relu(g2g_ref[...] * cen / jnp.sqrt(var + EPS) + g2b_ref[...])
    m = _colsum(hn) * (1.0 / N_NODES)
    mb = lax.dot_general(u_ref[...] * m[None, :], v_ref[...],
                         (((1,), (0,)), ((), ())),
                         precision=lax.Precision.HIGHEST,
                         preferred_element_type=jnp.float32)
    mu = jnp.mean(mb, axis=0)
    cb = mb - mu
    vb = jnp.mean(cb * cb, axis=0)
    out_ref[...] = n1g_ref[...] * cb / jnp.sqrt(vb + EPS) + n1b_ref[...]


def _final(parts, degp, b2t, g2gt, g2bt, n1_gamma, n1_beta):
    ks = jnp.arange(D)
    u = (ks[None, :] // G2 == jnp.arange(BATCH)[:, None]).astype(jnp.float32)
    v = (ks[:, None] % G2 == jnp.arange(G2)[None, :]).astype(jnp.float32)
    return pl.pallas_call(
        _final_body,
        out_shape=jax.ShapeDtypeStruct((BATCH, G2), jnp.float32),
    )(parts, degp, b2t, g2gt, g2bt, n1_gamma, n1_beta, u, v)


def kernel(x, edge_index, W1, b1, W2, b2, g1_gamma, g1_beta,
           g2_gamma, g2_beta, n1_gamma, n1_beta):
    src = edge_index[0]
    dst = edge_index[1]
    pad = jnp.full((E_PAD - N_EDGES,), N_NODES, jnp.int32)
    src_p = jnp.concatenate([src, pad])
    dst_p = jnp.concatenate([dst, pad])
    src2d = src_p.reshape(NW, NCH, CH)
    dst2d = dst_p.reshape(NW, NCH, CH)

    xr = x.reshape(BATCH, IN_DIM, N_NODES)

    deg_kernel, edge_kernel = _sc_kernels()
    degp = deg_kernel(src_p, dst_p)
    table1 = _project(xr, W1, degp)

    zrs = jnp.zeros((NROW, D), jnp.float32)
    parts1 = edge_kernel(table1, src2d, dst2d, zrs)

    eye = jnp.eye(BATCH, dtype=jnp.float32)
    W2blk = jnp.kron(eye, W2)
    b1t = jnp.tile(b1, BATCH)
    g1gt = jnp.tile(g1_gamma, BATCH)
    g1bt = jnp.tile(g1_beta, BATCH)
    table2 = _mid(parts1, degp, W2blk, b1t, g1gt, g1bt)

    parts2 = edge_kernel(table2, src2d, dst2d, zrs)

    b2t = jnp.tile(b2, BATCH)
    g2gt = jnp.tile(g2_gamma, BATCH)
    g2bt = jnp.tile(g2_beta, BATCH)
    return _final(parts2, degp, b2t, g2gt, g2bt, n1_gamma, n1_beta)


# speedup vs baseline: 15.0729x; 1.0245x over previous
import functools
import jax
import jax.numpy as jnp
from jax import lax
from jax.experimental import pallas as pl
from jax.experimental.pallas import tpu as pltpu
from jax.experimental.pallas import tpu_sc as plsc

N_NODES = 10000
N_EDGES = 320000
IN_DIM = 128
G1 = 10
G2 = 10
BATCH = 4
D = BATCH * G1
NROW = 10112
NC = 2
NS = 16
NW = NC * NS
CH = 128
NCH = 80
EPW = CH * NCH
E_PAD = EPW * NW
ROWS_PER_TILE = NROW // NS
EPS = 1e-5

def _deg_body(src_hbm, dst_hbm, out_hbm, sv, dv, hs, hd):
    w = lax.axis_index("s") * NC + lax.axis_index("c")
    base = w * EPW
    pltpu.sync_copy(src_hbm.at[pl.ds(base, EPW)], sv)
    pltpu.sync_copy(dst_hbm.at[pl.ds(base, EPW)], dv)

    zeros16 = jnp.zeros((16,), jnp.float32)

    def zero_body(i, _):
        hs[pl.ds(i * 16, 16)] = zeros16
        hd[pl.ds(i * 16, 16)] = zeros16
        return 0

    lax.fori_loop(0, NROW // 16, zero_body, 0, unroll=4)

    ones16 = jnp.ones((16,), jnp.float32)

    def hist_body(i, _):
        s16 = sv[pl.ds(i * 16, 16)]
        plsc.addupdate_scatter(hs, [s16], ones16)
        d16 = dv[pl.ds(i * 16, 16)]
        plsc.addupdate_scatter(hd, [d16], ones16)
        return 0

    lax.fori_loop(0, EPW // 16, hist_body, 0, unroll=4)

    pltpu.sync_copy(hs, out_hbm.at[w, 0])
    pltpu.sync_copy(hd, out_hbm.at[w, 1])


NSLOT = 8


def _edge_body(table_hbm, src_hbm, dst_hbm, zeros_hbm, out_hbm,
               acc, tbl, sv, dv,
               rb0, rb1, rb2, rb3, rb4, rb5, rb6, rb7,
               gs0, gs1, gs2, gs3, gs4, gs5, gs6, gs7,
               ss0, ss1, ss2, ss3, ss4, ss5, ss6, ss7):
    c = lax.axis_index("c")
    s = lax.axis_index("s")
    w = s * NC + c

    pltpu.sync_copy(zeros_hbm.at[pl.ds(s * ROWS_PER_TILE, ROWS_PER_TILE)],
                    acc.at[pl.ds(s * ROWS_PER_TILE, ROWS_PER_TILE)])
    pltpu.sync_copy(table_hbm.at[pl.ds(s * ROWS_PER_TILE, ROWS_PER_TILE)],
                    tbl.at[pl.ds(s * ROWS_PER_TILE, ROWS_PER_TILE)])
    plsc.subcore_barrier()

    pltpu.sync_copy(src_hbm.at[w], sv)
    pltpu.sync_copy(dst_hbm.at[w], dv)

    rbs = (rb0, rb1, rb2, rb3, rb4, rb5, rb6, rb7)
    gsems = (gs0, gs1, gs2, gs3, gs4, gs5, gs6, gs7)
    ssems = (ss0, ss1, ss2, ss3, ss4, ss5, ss6, ss7)
    for b in range(NSLOT):
        pltpu.async_copy(tbl.at[sv.at[b]], rbs[b], gsems[b])

    def body(i, _):
        k0 = NSLOT * i
        for b in range(NSLOT):
            k = k0 + b
            pltpu.make_async_copy(tbl.at[sv.at[k]], rbs[b],
                                  gsems[b]).wait()
            pltpu.async_copy(rbs[b], acc.at[dv.at[k]], ssems[b], add=True)
        for b in range(NSLOT):
            k = k0 + b
            pltpu.make_async_copy(rbs[b], acc.at[dv.at[k]], ssems[b]).wait()

            @pl.when(k + NSLOT < NCH)
            def _():
                pltpu.async_copy(tbl.at[sv.at[k + NSLOT]], rbs[b],
                                 gsems[b])
        return 0

    lax.fori_loop(0, NCH // NSLOT, body, 0)

    plsc.subcore_barrier()
    pltpu.sync_copy(acc.at[pl.ds(s * ROWS_PER_TILE, ROWS_PER_TILE)],
                    out_hbm.at[c, pl.ds(s * ROWS_PER_TILE, ROWS_PER_TILE)])


@functools.lru_cache(maxsize=None)
def _sc_kernels():
    mesh = plsc.VectorSubcoreMesh(core_axis_name="c", subcore_axis_name="s",
                                  num_cores=NC, num_subcores=NS)
    sc_params = pltpu.CompilerParams(needs_layout_passes=False,
                                     use_tc_tiling_on_sc=False)
    deg_kernel = pl.kernel(
        _deg_body,
        out_type=jax.ShapeDtypeStruct((NW, 2, NROW), jnp.float32),
        mesh=mesh,
        compiler_params=sc_params,
        scratch_types=[
            pltpu.VMEM((EPW,), jnp.int32),
            pltpu.VMEM((EPW,), jnp.int32),
            pltpu.VMEM((NROW,), jnp.float32),
            pltpu.VMEM((NROW,), jnp.float32),
        ],
    )
    edge_kernel = pl.kernel(
        _edge_body,
        out_type=jax.ShapeDtypeStruct((NC, NROW, D), jnp.float32),
        mesh=mesh,
        compiler_params=sc_params,
        scratch_types=[
            pltpu.VMEM_SHARED((NROW, D), jnp.float32),
            pltpu.VMEM_SHARED((NROW, D), jnp.float32),
            pltpu.VMEM((NCH, CH), jnp.int32),
            pltpu.VMEM((NCH, CH), jnp.int32),
        ] + [pltpu.VMEM((CH, D), jnp.float32)] * NSLOT
          + [pltpu.SemaphoreType.DMA] * (2 * NSLOT),
    )
    return deg_kernel, edge_kernel


def _proj_body(xr_ref, w1_ref, degp_ref, out_ref, norms_ref):
    deg = jnp.sum(degp_ref[...], axis=0)
    norms = lax.rsqrt(jnp.maximum(deg, 1.0))
    norms_ref[...] = norms
    nsrc = norms[0]
    w1 = w1_ref[...]
    outs = []
    for b in range(BATCH):
        xb = xr_ref[b] * nsrc[None, :]
        r = lax.dot_general(xb, w1, (((0,), (0,)), ((), ())),
                            preferred_element_type=jnp.float32)
        outs.append(r)
    out_ref[...] = jnp.concatenate(outs, axis=1)


_NB = 2048


def _project(xr, W1, degp):
    return pl.pallas_call(
        _proj_body,
        grid=(5,),
        in_specs=[
            pl.BlockSpec((BATCH, IN_DIM, _NB), lambda i: (0, 0, i)),
            pl.BlockSpec((IN_DIM, G1), lambda i: (0, 0)),
            pl.BlockSpec((NW, 2, _NB), lambda i: (0, 0, i)),
        ],
        out_specs=[pl.BlockSpec((_NB, D), lambda i: (i, 0)),
                   pl.BlockSpec((2, _NB), lambda i: (0, i))],
        out_shape=[jax.ShapeDtypeStruct((NROW, D), jnp.float32),
                   jax.ShapeDtypeStruct((2, NROW), jnp.float32)],
    )(xr, W1, degp)


def _colsum(h):
    n = h.shape[0]
    p = 1 << (n - 1).bit_length()
    if p != n:
        h = jnp.concatenate(
            [h, jnp.zeros((p - n,) + h.shape[1:], h.dtype)], axis=0)
    while p > 8:
        p //= 2
        h = h[:p] + h[p:]
    return jnp.sum(h, axis=0)


def _mid_body(parts_ref, norms_ref, w2b_ref, b1t_ref, g1g_ref, g1b_ref, out_ref):
    a = parts_ref[0, :N_NODES] + parts_ref[1, :N_NODES]
    nsrc = norms_ref[0, :N_NODES]
    ndst = norms_ref[1, :N_NODES]
    h = a * ndst[:, None] + b1t_ref[...]
    mean = _colsum(h) * (1.0 / N_NODES)
    cen = h - mean
    var = _colsum(cen * cen) * (1.0 / N_NODES)
    hn = jax.nn.relu(g1g_ref[...] * cen / jnp.sqrt(var + EPS) + g1b_ref[...])
    hn = hn * nsrc[:, None]
    h2 = lax.dot_general(hn, w2b_ref[...], (((1,), (0,)), ((), ())),
                         preferred_element_type=jnp.float32)
    out_ref[...] = jnp.concatenate(
        [h2, jnp.zeros((NROW - N_NODES, D), jnp.float32)], axis=0)


def _mid(parts, norms, W2blk, b1t, g1gt, g1bt):
    return pl.pallas_call(
        _mid_body,
        out_shape=jax.ShapeDtypeStruct((NROW, D), jnp.float32),
    )(parts, norms, W2blk, b1t, g1gt, g1bt)


def _final_body(parts_ref, norms_ref, b2t_ref, g2g_ref, g2b_ref,
                n1g_ref, n1b_ref, u_ref, v_ref, out_ref):
    a = parts_ref[0, :N_NODES] + parts_ref[1, :N_NODES]
    ndst = norms_ref[1, :N_NODES]
    h = a * ndst[:, None] + b2t_ref[...]
    mean = _colsum(h) * (1.0 / N_NODES)
    cen = h - mean
    var = _colsum(cen * cen) * (1.0 / N_NODES)
    hn = jax.nn.relu(g2g_ref[...] * cen / jnp.sqrt(var + EPS) + g2b_ref[...])
    m = _colsum(hn) * (1.0 / N_NODES)
    mb = lax.dot_general(u_ref[...] * m[None, :], v_ref[...],
                         (((1,), (0,)), ((), ())),
                         precision=lax.Precision.HIGHEST,
                         preferred_element_type=jnp.float32)
    mu = jnp.mean(mb, axis=0)
    cb = mb - mu
    vb = jnp.mean(cb * cb, axis=0)
    out_ref[...] = n1g_ref[...] * cb / jnp.sqrt(vb + EPS) + n1b_ref[...]


def _final(parts, norms, b2t, g2gt, g2bt, n1_gamma, n1_beta):
    ks = jnp.arange(D)
    u = (ks[None, :] // G2 == jnp.arange(BATCH)[:, None]).astype(jnp.float32)
    v = (ks[:, None] % G2 == jnp.arange(G2)[None, :]).astype(jnp.float32)
    return pl.pallas_call(
        _final_body,
        out_shape=jax.ShapeDtypeStruct((BATCH, G2), jnp.float32),
    )(parts, norms, b2t, g2gt, g2bt, n1_gamma, n1_beta, u, v)


def kernel(x, edge_index, W1, b1, W2, b2, g1_gamma, g1_beta,
           g2_gamma, g2_beta, n1_gamma, n1_beta):
    src = edge_index[0]
    dst = edge_index[1]
    pad = jnp.full((E_PAD - N_EDGES,), N_NODES, jnp.int32)
    src_p = jnp.concatenate([src, pad])
    dst_p = jnp.concatenate([dst, pad])
    src2d = src_p.reshape(NW, NCH, CH)
    dst2d = dst_p.reshape(NW, NCH, CH)

    xr = x.reshape(BATCH, IN_DIM, N_NODES)

    deg_kernel, edge_kernel = _sc_kernels()
    degp = deg_kernel(src_p, dst_p)
    table1, norms = _project(xr, W1, degp)

    zrs = jnp.zeros((NROW, D), jnp.float32)
    parts1 = edge_kernel(table1, src2d, dst2d, zrs)

    eye = jnp.eye(BATCH, dtype=jnp.float32)
    W2blk = jnp.kron(eye, W2)
    b1t = jnp.tile(b1, BATCH)
    g1gt = jnp.tile(g1_gamma, BATCH)
    g1bt = jnp.tile(g1_beta, BATCH)
    table2 = _mid(parts1, norms, W2blk, b1t, g1gt, g1bt)

    parts2 = edge_kernel(table2, src2d, dst2d, zrs)

    b2t = jnp.tile(b2, BATCH)
    g2gt = jnp.tile(g2_gamma, BATCH)
    g2bt = jnp.tile(g2_beta, BATCH)
    return _final(parts2, norms, b2t, g2gt, g2bt, n1_gamma, n1_beta)
